# Initial kernel scaffold; baseline (speedup 1.0000x reference)
import jax
import jax.numpy as jnp
from jax.experimental import pallas as pl


def kernel(z, supports, labels, weight, alpha_be, gamma_be, ensemble_bias):
    raise NotImplementedError("write your pallas kernel here")


# speedup vs baseline: 1.2203x; 1.2203x over previous
import functools

import jax
import jax.numpy as jnp
from jax import lax
from jax.experimental import pallas as pl
from jax.experimental.pallas import tpu as pltpu
from jax.experimental.pallas import tpu_sc as plsc

B, N, D, C, E = 1024, 20000, 64, 17, 5
TAU, K = 10.0, 20
DP = 128
CP = 32
NP = 20480
NA = 2048
BB = 128
NEG_BIG = -1e30


def _rownorm(x):
    n = jnp.sqrt(jnp.sum(x * x, axis=1, keepdims=True))
    return x / jnp.maximum(n, 1e-12)


def _stage1a_body(sup_ref, lab_ref, alpha_ref, gamma_ref, bias_ref, w_ref,
                  u_ref, cnt_ref):
    @pl.when(pl.program_id(0) == 0)
    def _init():
        u_ref[...] = jnp.zeros_like(u_ref)
        cnt_ref[...] = jnp.zeros_like(cnt_ref)

    sup = sup_ref[...]
    lab = lab_ref[...]
    w = w_ref[...]
    ones = jnp.ones((NA, DP), jnp.float32)
    cnt_ref[...] += lax.dot_general(lab, ones, (((0,), (0,)), ((), ())),
                                    preferred_element_type=jnp.float32)
    for e in range(E):
        r = sup * alpha_ref[e:e + 1, :]
        mlp = lax.dot_general(r, w, (((1,), (1,)), ((), ())),
                              preferred_element_type=jnp.float32)
        mlp = mlp * gamma_ref[e:e + 1, :] + bias_ref[e:e + 1, :]
        u_ref[e * CP:(e + 1) * CP, :] += lax.dot_general(
            lab, mlp, (((0,), (0,)), ((), ())),
            preferred_element_type=jnp.float32)


def _stage1b_body(sup_ref, alpha_ref, gamma_ref, bias_ref, w_ref,
                  u_ref, cnt_ref, q_ref):
    sup = sup_ref[...]
    w = w_ref[...]
    lane_c = lax.broadcasted_iota(jnp.int32, (NA, CP), 1)
    qacc = jnp.zeros((NA, CP), jnp.float32)
    for e in range(E):
        r = sup * alpha_ref[e:e + 1, :]
        mlp = lax.dot_general(r, w, (((1,), (1,)), ((), ())),
                              preferred_element_type=jnp.float32)
        mlp = mlp * gamma_ref[e:e + 1, :] + bias_ref[e:e + 1, :]
        tz = _rownorm(mlp)
        cen = u_ref[e * CP:(e + 1) * CP, :] / (cnt_ref[...] + 1e-12)
        cenn = _rownorm(cen)
        logits = TAU * lax.dot_general(tz, cenn, (((1,), (1,)), ((), ())),
                                       preferred_element_type=jnp.float32)
        logits = jnp.where(lane_c < C, logits, NEG_BIG)
        m = jnp.max(logits, axis=1, keepdims=True)
        p = jnp.exp(logits - m)
        qacc = qacc + p / jnp.sum(p, axis=1, keepdims=True)
    q_ref[...] = jnp.concatenate(
        [qacc * (1.0 / (E * K)), jnp.zeros((NA, DP - CP), jnp.float32)],
        axis=1)


NCH = NP // NA


def _stage2_body(z_ref, sup_ref, idx_ref, wb3):
    zb = z_ref[...]
    x = _rownorm(zb)
    lane_d = lax.broadcasted_iota(jnp.int32, (BB, DP), 1)
    xa = jnp.where(lane_d == D, 1.0, -2.0 * x)
    for j in range(NCH):
        yc = sup_ref[j * NA:(j + 1) * NA, :]
        yn = _rownorm(yc)
        yy = jnp.sum(yn * yn, axis=1, keepdims=True)
        rowid = j * NA + lax.broadcasted_iota(jnp.int32, (NA, 1), 0)
        yy = jnp.where(rowid < N, yy, 1e30)
        lane_y = lax.broadcasted_iota(jnp.int32, (NA, DP), 1)
        ya = jnp.where(lane_y == D, yy, yn)
        wb3[j] = lax.dot_general(xa, ya, (((1,), (1,)), ((), ())),
                                 preferred_element_type=jnp.float32)

    idx_ref[...] = jnp.zeros((BB, 128), jnp.int32)
    lane_c = lax.broadcasted_iota(jnp.int32, (BB, NA), 1)
    big_i = jnp.int32(2**30)
    for k in range(K):
        def mn_body(j, cur):
            return jnp.minimum(cur, jnp.min(wb3[j], axis=1, keepdims=True))
        mn = lax.fori_loop(0, NCH, mn_body,
                           jnp.full((BB, 1), jnp.inf, jnp.float32))

        def ix_body(j, cur):
            cand = jnp.where(wb3[j] == mn, lane_c + j * NA, big_i)
            return jnp.minimum(cur, jnp.min(cand, axis=1, keepdims=True))
        sel = lax.fori_loop(0, NCH, ix_body, jnp.full((BB, 1), big_i))

        idx_ref[:, k:k + 1] = sel

        def up_body(j, _):
            g = lane_c + j * NA
            wb3[j] = jnp.where(g == sel, jnp.float32(1e30), wb3[j])
            return 0
        lax.fori_loop(0, NCH, up_body, 0)


_NC, _NS = 2, 16
_NW = _NC * _NS
_QPW = B // _NW
_IPW = _QPW * K
_ICH = _IPW // 128


def _sc_gather_body(q_hbm, idx_hbm, out_hbm, idx_v, rows_v, acc_v, sem):
    wid = lax.axis_index("s") * _NC + lax.axis_index("c")
    for j in range(_ICH):
        pltpu.sync_copy(idx_hbm.at[pl.ds(wid * _IPW + j * 128, 128)],
                        idx_v.at[j])
    copies = [
        pltpu.async_copy(q_hbm.at[idx_v.at[j]],
                         rows_v.at[pl.ds(j * 128, 128)], sem)
        for j in range(_ICH)
    ]
    for cp in copies:
        cp.wait()

    def qbody(i, _):
        a0 = jnp.zeros((16,), jnp.float32)
        a1 = jnp.zeros((16,), jnp.float32)
        for j in range(K):
            a0 = a0 + rows_v[i * K + j, pl.ds(0, 16)]
            a1 = a1 + rows_v[i * K + j, pl.ds(16, 16)]
        acc_v[i, pl.ds(0, 16)] = a0
        acc_v[i, pl.ds(16, 16)] = a1
        z16 = jnp.zeros((16,), jnp.float32)
        for h in range(2, 8):
            acc_v[i, pl.ds(h * 16, 16)] = z16
        return 0

    lax.fori_loop(0, _QPW, qbody, 0)
    pltpu.sync_copy(acc_v, out_hbm.at[pl.ds(wid * _QPW, _QPW)])


@functools.cache
def _build_sc_gather():
    return functools.partial(
        pl.kernel,
        out_type=jax.ShapeDtypeStruct((B, DP), jnp.float32),
        mesh=plsc.VectorSubcoreMesh(core_axis_name="c", subcore_axis_name="s"),
        scratch_types=[
            pltpu.VMEM((_ICH, 128), jnp.int32),
            pltpu.VMEM((_IPW, DP), jnp.float32),
            pltpu.VMEM((_QPW, DP), jnp.float32),
            pltpu.SemaphoreType.DMA,
        ],
    )(_sc_gather_body)


def kernel(z, supports, labels, weight, alpha_be, gamma_be, ensemble_bias):
    f32 = jnp.float32
    zp = jnp.pad(z.astype(f32), ((0, 0), (0, DP - D)))
    supp = jnp.pad(supports.astype(f32), ((0, NP - N), (0, DP - D)))
    labp = jnp.pad(labels.astype(f32), ((0, NP - N), (0, CP - C)))
    wp = jnp.pad(weight.astype(f32), ((0, DP - D), (0, DP - D)))
    ap = jnp.pad(alpha_be.astype(f32), ((0, 8 - E), (0, DP - D)))
    gp = jnp.pad(gamma_be.astype(f32), ((0, 8 - E), (0, DP - D)))
    bp = jnp.pad(ensemble_bias.astype(f32), ((0, 8 - E), (0, DP - D)))

    grid1 = (NP // NA,)
    full = lambda shape: pl.BlockSpec(shape, lambda i: (0, 0))
    u, cnt = pl.pallas_call(
        _stage1a_body,
        grid=grid1,
        in_specs=[
            pl.BlockSpec((NA, DP), lambda i: (i, 0)),
            pl.BlockSpec((NA, CP), lambda i: (i, 0)),
            full((8, DP)), full((8, DP)), full((8, DP)), full((DP, DP)),
        ],
        out_specs=(full((E * CP, DP)), full((CP, DP))),
        out_shape=(jax.ShapeDtypeStruct((E * CP, DP), f32),
                   jax.ShapeDtypeStruct((CP, DP), f32)),
    )(supp, labp, ap, gp, bp, wp)

    q = pl.pallas_call(
        _stage1b_body,
        grid=grid1,
        in_specs=[
            pl.BlockSpec((NA, DP), lambda i: (i, 0)),
            full((8, DP)), full((8, DP)), full((8, DP)), full((DP, DP)),
            full((E * CP, DP)), full((CP, DP)),
        ],
        out_specs=pl.BlockSpec((NA, DP), lambda i: (i, 0)),
        out_shape=jax.ShapeDtypeStruct((NP, DP), f32),
    )(supp, ap, gp, bp, wp, u, cnt)

    idx = pl.pallas_call(
        _stage2_body,
        grid=(B // BB,),
        in_specs=[
            pl.BlockSpec((BB, DP), lambda i: (i, 0)),
            pl.BlockSpec((NP, DP), lambda i: (0, 0)),
        ],
        out_specs=pl.BlockSpec((BB, 128), lambda i: (i, 0)),
        out_shape=jax.ShapeDtypeStruct((B, 128), jnp.int32),
        scratch_shapes=[pltpu.VMEM((NCH, BB, NA), f32)],
    )(zp, supp)

    idx_flat = idx[:, :K].reshape(B * K)
    out = _build_sc_gather()(q, idx_flat)
    return out[:, :C]


# speedup vs baseline: 1.3295x; 1.0895x over previous
import functools

import jax
import jax.numpy as jnp
from jax import lax
from jax.experimental import pallas as pl
from jax.experimental.pallas import tpu as pltpu
from jax.experimental.pallas import tpu_sc as plsc

B, N, D, C, E = 1024, 20000, 64, 17, 5
TAU, K = 10.0, 20
DP = 128
CP = 32
NP = 20480
NA = 2048
BB = 128
NEG_BIG = -1e30


def _rownorm(x):
    n = jnp.sqrt(jnp.sum(x * x, axis=1, keepdims=True))
    return x / jnp.maximum(n, 1e-12)


def _stage1a_body(sup_ref, lab_ref, alpha_ref, gamma_ref, bias_ref, w_ref,
                  u_ref, cnt_ref):
    @pl.when(pl.program_id(0) == 0)
    def _init():
        u_ref[...] = jnp.zeros_like(u_ref)
        cnt_ref[...] = jnp.zeros_like(cnt_ref)

    sup = sup_ref[...]
    lab = lab_ref[...]
    w = w_ref[...]
    ones = jnp.ones((NA, DP), jnp.float32)
    cnt_ref[...] += lax.dot_general(lab, ones, (((0,), (0,)), ((), ())),
                                    preferred_element_type=jnp.float32)
    for e in range(E):
        r = sup * alpha_ref[e:e + 1, :]
        mlp = lax.dot_general(r, w, (((1,), (1,)), ((), ())),
                              preferred_element_type=jnp.float32)
        mlp = mlp * gamma_ref[e:e + 1, :] + bias_ref[e:e + 1, :]
        u_ref[e * CP:(e + 1) * CP, :] += lax.dot_general(
            lab, mlp, (((0,), (0,)), ((), ())),
            preferred_element_type=jnp.float32)


def _stage1b_body(sup_ref, alpha_ref, gamma_ref, bias_ref, w_ref,
                  u_ref, cnt_ref, q_ref, ya_ref):
    sup = sup_ref[...]
    w = w_ref[...]
    yn = _rownorm(sup)
    yy = jnp.sum(yn * yn, axis=1, keepdims=True)
    rowid = pl.program_id(0) * NA + lax.broadcasted_iota(jnp.int32, (NA, 1), 0)
    yy = jnp.where(rowid < N, yy, 1e30)
    lane_y = lax.broadcasted_iota(jnp.int32, (NA, DP), 1)
    ya_ref[...] = jnp.where(lane_y == D, yy, yn)
    lane_c = lax.broadcasted_iota(jnp.int32, (NA, CP), 1)
    qacc = jnp.zeros((NA, CP), jnp.float32)
    for e in range(E):
        r = sup * alpha_ref[e:e + 1, :]
        mlp = lax.dot_general(r, w, (((1,), (1,)), ((), ())),
                              preferred_element_type=jnp.float32)
        mlp = mlp * gamma_ref[e:e + 1, :] + bias_ref[e:e + 1, :]
        tz = _rownorm(mlp)
        cen = u_ref[e * CP:(e + 1) * CP, :] / (cnt_ref[...] + 1e-12)
        cenn = _rownorm(cen)
        logits = TAU * lax.dot_general(tz, cenn, (((1,), (1,)), ((), ())),
                                       preferred_element_type=jnp.float32)
        logits = jnp.where(lane_c < C, logits, NEG_BIG)
        m = jnp.max(logits, axis=1, keepdims=True)
        p = jnp.exp(logits - m)
        qacc = qacc + p / jnp.sum(p, axis=1, keepdims=True)
    q_ref[...] = jnp.concatenate(
        [qacc * (1.0 / (E * K)), jnp.zeros((NA, DP - CP), jnp.float32)],
        axis=1)


NCH = NP // NA


def _stage2_body(z_ref, ya_ref, idx_ref, wb3):
    zb = z_ref[...]
    x = _rownorm(zb)
    lane_d = lax.broadcasted_iota(jnp.int32, (BB, DP), 1)
    xa = jnp.where(lane_d == D, 1.0, -2.0 * x)
    for j in range(NCH):
        wb3[j] = lax.dot_general(xa, ya_ref[j * NA:(j + 1) * NA, :],
                                 (((1,), (1,)), ((), ())),
                                 preferred_element_type=jnp.float32)

    idx_ref[...] = jnp.zeros((BB, 128), jnp.int32)
    lane_c = lax.broadcasted_iota(jnp.int32, (BB, NA), 1)
    big_i = jnp.int32(2**30)

    def mn_body(j, cur):
        return jnp.minimum(cur, jnp.min(wb3[j], axis=1, keepdims=True))
    mn = lax.fori_loop(0, NCH, mn_body,
                       jnp.full((BB, 1), jnp.inf, jnp.float32))

    for k in range(K):
        def ix_body(j, cur):
            cand = jnp.where(wb3[j] == mn, lane_c + j * NA, big_i)
            return jnp.minimum(cur, jnp.min(cand, axis=1, keepdims=True))
        sel = lax.fori_loop(0, NCH, ix_body, jnp.full((BB, 1), big_i))

        idx_ref[:, k:k + 1] = sel

        if k < K - 1:
            def up_body(j, cur):
                g = lane_c + j * NA
                neww = jnp.where(g == sel, jnp.float32(1e30), wb3[j])
                wb3[j] = neww
                return jnp.minimum(cur, jnp.min(neww, axis=1, keepdims=True))
            mn = lax.fori_loop(0, NCH, up_body,
                               jnp.full((BB, 1), jnp.inf, jnp.float32))


_NC, _NS = 2, 16
_NW = _NC * _NS
_QPW = B // _NW
_IPW = _QPW * K
_ICH = _IPW // 128


def _sc_gather_body(q_hbm, idx_hbm, out_hbm, idx_v, rows_v, acc_v, sem):
    wid = lax.axis_index("s") * _NC + lax.axis_index("c")
    for j in range(_ICH):
        pltpu.sync_copy(idx_hbm.at[pl.ds(wid * _IPW + j * 128, 128)],
                        idx_v.at[j])
    copies = [
        pltpu.async_copy(q_hbm.at[idx_v.at[j]],
                         rows_v.at[pl.ds(j * 128, 128)], sem)
        for j in range(_ICH)
    ]
    for cp in copies:
        cp.wait()

    def qbody(i, _):
        a0 = jnp.zeros((16,), jnp.float32)
        a1 = jnp.zeros((16,), jnp.float32)
        for j in range(K):
            a0 = a0 + rows_v[i * K + j, pl.ds(0, 16)]
            a1 = a1 + rows_v[i * K + j, pl.ds(16, 16)]
        acc_v[i, pl.ds(0, 16)] = a0
        acc_v[i, pl.ds(16, 16)] = a1
        z16 = jnp.zeros((16,), jnp.float32)
        for h in range(2, 8):
            acc_v[i, pl.ds(h * 16, 16)] = z16
        return 0

    lax.fori_loop(0, _QPW, qbody, 0)
    pltpu.sync_copy(acc_v, out_hbm.at[pl.ds(wid * _QPW, _QPW)])


@functools.cache
def _build_sc_gather():
    return functools.partial(
        pl.kernel,
        out_type=jax.ShapeDtypeStruct((B, DP), jnp.float32),
        mesh=plsc.VectorSubcoreMesh(core_axis_name="c", subcore_axis_name="s"),
        scratch_types=[
            pltpu.VMEM((_ICH, 128), jnp.int32),
            pltpu.VMEM((_IPW, DP), jnp.float32),
            pltpu.VMEM((_QPW, DP), jnp.float32),
            pltpu.SemaphoreType.DMA,
        ],
    )(_sc_gather_body)


def kernel(z, supports, labels, weight, alpha_be, gamma_be, ensemble_bias):
    f32 = jnp.float32
    zp = jnp.pad(z.astype(f32), ((0, 0), (0, DP - D)))
    supp = jnp.pad(supports.astype(f32), ((0, NP - N), (0, DP - D)))
    labp = jnp.pad(labels.astype(f32), ((0, NP - N), (0, CP - C)))
    wp = jnp.pad(weight.astype(f32), ((0, DP - D), (0, DP - D)))
    ap = jnp.pad(alpha_be.astype(f32), ((0, 8 - E), (0, DP - D)))
    gp = jnp.pad(gamma_be.astype(f32), ((0, 8 - E), (0, DP - D)))
    bp = jnp.pad(ensemble_bias.astype(f32), ((0, 8 - E), (0, DP - D)))

    grid1 = (NP // NA,)
    full = lambda shape: pl.BlockSpec(shape, lambda i: (0, 0))
    u, cnt = pl.pallas_call(
        _stage1a_body,
        grid=grid1,
        in_specs=[
            pl.BlockSpec((NA, DP), lambda i: (i, 0)),
            pl.BlockSpec((NA, CP), lambda i: (i, 0)),
            full((8, DP)), full((8, DP)), full((8, DP)), full((DP, DP)),
        ],
        out_specs=(full((E * CP, DP)), full((CP, DP))),
        out_shape=(jax.ShapeDtypeStruct((E * CP, DP), f32),
                   jax.ShapeDtypeStruct((CP, DP), f32)),
    )(supp, labp, ap, gp, bp, wp)

    q, ya = pl.pallas_call(
        _stage1b_body,
        grid=grid1,
        in_specs=[
            pl.BlockSpec((NA, DP), lambda i: (i, 0)),
            full((8, DP)), full((8, DP)), full((8, DP)), full((DP, DP)),
            full((E * CP, DP)), full((CP, DP)),
        ],
        out_specs=(pl.BlockSpec((NA, DP), lambda i: (i, 0)),
                   pl.BlockSpec((NA, DP), lambda i: (i, 0))),
        out_shape=(jax.ShapeDtypeStruct((NP, DP), f32),
                   jax.ShapeDtypeStruct((NP, DP), f32)),
    )(supp, ap, gp, bp, wp, u, cnt)

    idx = pl.pallas_call(
        _stage2_body,
        grid=(B // BB,),
        in_specs=[
            pl.BlockSpec((BB, DP), lambda i: (i, 0)),
            pl.BlockSpec((NP, DP), lambda i: (0, 0)),
        ],
        out_specs=pl.BlockSpec((BB, 128), lambda i: (i, 0)),
        out_shape=jax.ShapeDtypeStruct((B, 128), jnp.int32),
        scratch_shapes=[pltpu.VMEM((NCH, BB, NA), f32)],
    )(zp, ya)

    idx_flat = idx[:, :K].reshape(B * K)
    out = _build_sc_gather()(q, idx_flat)
    return out[:, :C]


# speedup vs baseline: 1.4379x; 1.0815x over previous
import functools

import jax
import jax.numpy as jnp
from jax import lax
from jax.experimental import pallas as pl
from jax.experimental.pallas import tpu as pltpu
from jax.experimental.pallas import tpu_sc as plsc

B, N, D, C, E = 1024, 20000, 64, 17, 5
TAU, K = 10.0, 20
DP = 128
CP = 32
NP = 20480
NA = 2048
BB = 128
NEG_BIG = -1e30


def _rownorm(x):
    n = jnp.sqrt(jnp.sum(x * x, axis=1, keepdims=True))
    return x / jnp.maximum(n, 1e-12)


def _stage1a_body(sup_ref, lab_ref, alpha_ref, gamma_ref, bias_ref, w_ref,
                  u_ref, cnt_ref):
    @pl.when(pl.program_id(0) == 0)
    def _init():
        u_ref[...] = jnp.zeros_like(u_ref)
        cnt_ref[...] = jnp.zeros_like(cnt_ref)

    sup = sup_ref[...]
    lab = lab_ref[...]
    w = w_ref[...]
    ones = jnp.ones((NA, DP), jnp.float32)
    cnt_ref[...] += lax.dot_general(lab, ones, (((0,), (0,)), ((), ())),
                                    preferred_element_type=jnp.float32)
    for e in range(E):
        r = sup * alpha_ref[e:e + 1, :]
        mlp = lax.dot_general(r, w, (((1,), (1,)), ((), ())),
                              preferred_element_type=jnp.float32)
        mlp = mlp * gamma_ref[e:e + 1, :] + bias_ref[e:e + 1, :]
        u_ref[e * CP:(e + 1) * CP, :] += lax.dot_general(
            lab, mlp, (((0,), (0,)), ((), ())),
            preferred_element_type=jnp.float32)


def _stage1b_body(sup_ref, alpha_ref, gamma_ref, bias_ref, w_ref,
                  u_ref, cnt_ref, q_ref, ya_ref):
    sup = sup_ref[...]
    w = w_ref[...]
    yn = _rownorm(sup)
    yy = jnp.sum(yn * yn, axis=1, keepdims=True)
    rowid = pl.program_id(0) * NA + lax.broadcasted_iota(jnp.int32, (NA, 1), 0)
    yy = jnp.where(rowid < N, yy, 1e30)
    lane_y = lax.broadcasted_iota(jnp.int32, (NA, DP), 1)
    ya_ref[...] = jnp.where(lane_y == D, yy, yn)
    lane_c = lax.broadcasted_iota(jnp.int32, (NA, CP), 1)
    qacc = jnp.zeros((NA, CP), jnp.float32)
    for e in range(E):
        r = sup * alpha_ref[e:e + 1, :]
        mlp = lax.dot_general(r, w, (((1,), (1,)), ((), ())),
                              preferred_element_type=jnp.float32)
        mlp = mlp * gamma_ref[e:e + 1, :] + bias_ref[e:e + 1, :]
        tz = _rownorm(mlp)
        cen = u_ref[e * CP:(e + 1) * CP, :] / (cnt_ref[...] + 1e-12)
        cenn = _rownorm(cen)
        logits = TAU * lax.dot_general(tz, cenn, (((1,), (1,)), ((), ())),
                                       preferred_element_type=jnp.float32)
        logits = jnp.where(lane_c < C, logits, NEG_BIG)
        m = jnp.max(logits, axis=1, keepdims=True)
        p = jnp.exp(logits - m)
        qacc = qacc + p / jnp.sum(p, axis=1, keepdims=True)
    q_ref[...] = jnp.concatenate(
        [qacc * (1.0 / (E * K)), jnp.zeros((NA, DP - CP), jnp.float32)],
        axis=1)


NCH = NP // NA


def _stage2_body(z_ref, ya_ref, idx_ref, wb3):
    zb = z_ref[...]
    x = _rownorm(zb)
    lane_d = lax.broadcasted_iota(jnp.int32, (BB, DP), 1)
    xa = jnp.where(lane_d == D, 1.0, -2.0 * x)
    for j in range(NCH):
        wb3[j] = lax.dot_general(xa, ya_ref[j * NA:(j + 1) * NA, :],
                                 (((1,), (1,)), ((), ())),
                                 preferred_element_type=jnp.float32)

    idx_ref[...] = jnp.zeros((BB, 128), jnp.int32)
    lane_c = lax.broadcasted_iota(jnp.int32, (BB, NA), 1)
    big_i = jnp.int32(2**30)

    def mn_body(j, cur):
        return jnp.minimum(cur, jnp.min(wb3[j], axis=1, keepdims=True))
    mn = lax.fori_loop(0, NCH, mn_body,
                       jnp.full((BB, 1), jnp.inf, jnp.float32))

    for k in range(K):
        def sweep(j, carry):
            cur, nxt = carry
            w = wb3[j]
            candj = jnp.min(jnp.where(w == mn, lane_c, big_i),
                            axis=1, keepdims=True)
            local_win = jnp.logical_and(cur == big_i, candj < big_i)
            clearmask = jnp.logical_and(local_win, lane_c == candj)
            neww = jnp.where(clearmask, jnp.float32(1e30), w)
            wb3[j] = neww
            cur = jnp.where(local_win, candj + j * NA, cur)
            nxt = jnp.minimum(nxt, jnp.min(neww, axis=1, keepdims=True))
            return cur, nxt

        sel, mn = lax.fori_loop(
            0, NCH, sweep,
            (jnp.full((BB, 1), big_i),
             jnp.full((BB, 1), jnp.inf, jnp.float32)))
        idx_ref[:, k:k + 1] = sel


_NC, _NS = 2, 16
_NW = _NC * _NS
_QPW = B // _NW
_IPW = _QPW * K
_ICH = _IPW // 128


def _sc_gather_body(q_hbm, idx_hbm, out_hbm, idx_v, rows_v, acc_v, sem):
    wid = lax.axis_index("s") * _NC + lax.axis_index("c")
    for j in range(_ICH):
        pltpu.sync_copy(idx_hbm.at[pl.ds(wid * _IPW + j * 128, 128)],
                        idx_v.at[j])
    copies = [
        pltpu.async_copy(q_hbm.at[idx_v.at[j]],
                         rows_v.at[pl.ds(j * 128, 128)], sem)
        for j in range(_ICH)
    ]
    for cp in copies:
        cp.wait()

    def qbody(i, _):
        a0 = jnp.zeros((16,), jnp.float32)
        a1 = jnp.zeros((16,), jnp.float32)
        for j in range(K):
            a0 = a0 + rows_v[i * K + j, pl.ds(0, 16)]
            a1 = a1 + rows_v[i * K + j, pl.ds(16, 16)]
        acc_v[i, pl.ds(0, 16)] = a0
        acc_v[i, pl.ds(16, 16)] = a1
        z16 = jnp.zeros((16,), jnp.float32)
        for h in range(2, 8):
            acc_v[i, pl.ds(h * 16, 16)] = z16
        return 0

    lax.fori_loop(0, _QPW, qbody, 0)
    pltpu.sync_copy(acc_v, out_hbm.at[pl.ds(wid * _QPW, _QPW)])


@functools.cache
def _build_sc_gather():
    return functools.partial(
        pl.kernel,
        out_type=jax.ShapeDtypeStruct((B, DP), jnp.float32),
        mesh=plsc.VectorSubcoreMesh(core_axis_name="c", subcore_axis_name="s"),
        scratch_types=[
            pltpu.VMEM((_ICH, 128), jnp.int32),
            pltpu.VMEM((_IPW, DP), jnp.float32),
            pltpu.VMEM((_QPW, DP), jnp.float32),
            pltpu.SemaphoreType.DMA,
        ],
    )(_sc_gather_body)


def kernel(z, supports, labels, weight, alpha_be, gamma_be, ensemble_bias):
    f32 = jnp.float32
    zp = jnp.pad(z.astype(f32), ((0, 0), (0, DP - D)))
    supp = jnp.pad(supports.astype(f32), ((0, NP - N), (0, DP - D)))
    labp = jnp.pad(labels.astype(f32), ((0, NP - N), (0, CP - C)))
    wp = jnp.pad(weight.astype(f32), ((0, DP - D), (0, DP - D)))
    ap = jnp.pad(alpha_be.astype(f32), ((0, 8 - E), (0, DP - D)))
    gp = jnp.pad(gamma_be.astype(f32), ((0, 8 - E), (0, DP - D)))
    bp = jnp.pad(ensemble_bias.astype(f32), ((0, 8 - E), (0, DP - D)))

    grid1 = (NP // NA,)
    full = lambda shape: pl.BlockSpec(shape, lambda i: (0, 0))
    u, cnt = pl.pallas_call(
        _stage1a_body,
        grid=grid1,
        in_specs=[
            pl.BlockSpec((NA, DP), lambda i: (i, 0)),
            pl.BlockSpec((NA, CP), lambda i: (i, 0)),
            full((8, DP)), full((8, DP)), full((8, DP)), full((DP, DP)),
        ],
        out_specs=(full((E * CP, DP)), full((CP, DP))),
        out_shape=(jax.ShapeDtypeStruct((E * CP, DP), f32),
                   jax.ShapeDtypeStruct((CP, DP), f32)),
    )(supp, labp, ap, gp, bp, wp)

    q, ya = pl.pallas_call(
        _stage1b_body,
        grid=grid1,
        in_specs=[
            pl.BlockSpec((NA, DP), lambda i: (i, 0)),
            full((8, DP)), full((8, DP)), full((8, DP)), full((DP, DP)),
            full((E * CP, DP)), full((CP, DP)),
        ],
        out_specs=(pl.BlockSpec((NA, DP), lambda i: (i, 0)),
                   pl.BlockSpec((NA, DP), lambda i: (i, 0))),
        out_shape=(jax.ShapeDtypeStruct((NP, DP), f32),
                   jax.ShapeDtypeStruct((NP, DP), f32)),
    )(supp, ap, gp, bp, wp, u, cnt)

    idx = pl.pallas_call(
        _stage2_body,
        grid=(B // BB,),
        in_specs=[
            pl.BlockSpec((BB, DP), lambda i: (i, 0)),
            pl.BlockSpec((NP, DP), lambda i: (0, 0)),
        ],
        out_specs=pl.BlockSpec((BB, 128), lambda i: (i, 0)),
        out_shape=jax.ShapeDtypeStruct((B, 128), jnp.int32),
        scratch_shapes=[pltpu.VMEM((NCH, BB, NA), f32)],
    )(zp, ya)

    idx_flat = idx[:, :K].reshape(B * K)
    out = _build_sc_gather()(q, idx_flat)
    return out[:, :C]


# speedup vs baseline: 1.8416x; 1.2807x over previous
import functools

import jax
import jax.numpy as jnp
from jax import lax
from jax.experimental import pallas as pl
from jax.experimental.pallas import tpu as pltpu
from jax.experimental.pallas import tpu_sc as plsc

B, N, D, C, E = 1024, 20000, 64, 17, 5
TAU, K = 10.0, 20
DP = 128
CP = 32
NP = 20480
NA = 2048
BB = 128
NEG_BIG = -1e30


def _rownorm(x):
    n = jnp.sqrt(jnp.sum(x * x, axis=1, keepdims=True))
    return x / jnp.maximum(n, 1e-12)


def _stage1a_body(sup_ref, lab_ref, alpha_ref, gamma_ref, bias_ref, w_ref,
                  u_ref, cnt_ref):
    @pl.when(pl.program_id(0) == 0)
    def _init():
        u_ref[...] = jnp.zeros_like(u_ref)
        cnt_ref[...] = jnp.zeros_like(cnt_ref)

    sup = sup_ref[...]
    lab = lab_ref[...]
    w = w_ref[...]
    ones = jnp.ones((NA, DP), jnp.float32)
    cnt_ref[...] += lax.dot_general(lab, ones, (((0,), (0,)), ((), ())),
                                    preferred_element_type=jnp.float32)
    for e in range(E):
        r = sup * alpha_ref[e:e + 1, :]
        mlp = lax.dot_general(r, w, (((1,), (1,)), ((), ())),
                              preferred_element_type=jnp.float32)
        mlp = mlp * gamma_ref[e:e + 1, :] + bias_ref[e:e + 1, :]
        u_ref[e * CP:(e + 1) * CP, :] += lax.dot_general(
            lab, mlp, (((0,), (0,)), ((), ())),
            preferred_element_type=jnp.float32)


def _stage1b_body(sup_ref, alpha_ref, gamma_ref, bias_ref, w_ref,
                  u_ref, cnt_ref, q_ref, ya_ref):
    sup = sup_ref[...]
    w = w_ref[...]
    yn = _rownorm(sup)
    yy = jnp.sum(yn * yn, axis=1, keepdims=True)
    rowid = pl.program_id(0) * NA + lax.broadcasted_iota(jnp.int32, (NA, 1), 0)
    yy = jnp.where(rowid < N, yy, 1e30)
    lane_y = lax.broadcasted_iota(jnp.int32, (NA, DP), 1)
    ya_ref[...] = jnp.where(lane_y == D, yy, yn)
    lane_c = lax.broadcasted_iota(jnp.int32, (NA, CP), 1)
    qacc = jnp.zeros((NA, CP), jnp.float32)
    for e in range(E):
        r = sup * alpha_ref[e:e + 1, :]
        mlp = lax.dot_general(r, w, (((1,), (1,)), ((), ())),
                              preferred_element_type=jnp.float32)
        mlp = mlp * gamma_ref[e:e + 1, :] + bias_ref[e:e + 1, :]
        tz = _rownorm(mlp)
        cen = u_ref[e * CP:(e + 1) * CP, :] / (cnt_ref[...] + 1e-12)
        cenn = _rownorm(cen)
        logits = TAU * lax.dot_general(tz, cenn, (((1,), (1,)), ((), ())),
                                       preferred_element_type=jnp.float32)
        logits = jnp.where(lane_c < C, logits, NEG_BIG)
        m = jnp.max(logits, axis=1, keepdims=True)
        p = jnp.exp(logits - m)
        qacc = qacc + p / jnp.sum(p, axis=1, keepdims=True)
    q_ref[...] = jnp.concatenate(
        [qacc * (1.0 / (E * K)), jnp.zeros((NA, DP - CP), jnp.float32)],
        axis=1)


NCH = 2
NB = NP // NCH


def _stage2_body(z_ref, ya_ref, idx_ref, wb3):
    zb = z_ref[...]
    x = _rownorm(zb)
    lane_d = lax.broadcasted_iota(jnp.int32, (BB, DP), 1)
    xa = jnp.where(lane_d == D, 1.0, -2.0 * x)
    for j in range(NCH):
        for t in range(NB // NA):
            wb3[j, :, t * NA:(t + 1) * NA] = lax.dot_general(
                xa, ya_ref[j * NB + t * NA:j * NB + (t + 1) * NA, :],
                (((1,), (1,)), ((), ())),
                preferred_element_type=jnp.float32)

    lane_c = lax.broadcasted_iota(jnp.int32, (BB, NB), 1)
    lane_k = lax.broadcasted_iota(jnp.int32, (BB, 128), 1)
    big_i = jnp.int32(2**30)

    def mn_body(j, cur):
        return jnp.minimum(cur, jnp.min(wb3[j], axis=1, keepdims=True))
    mn0 = lax.fori_loop(0, NCH, mn_body,
                        jnp.full((BB, 1), jnp.inf, jnp.float32))

    def kbody(k, carry):
        mn, sels = carry

        def sweep(j, c2):
            cur, nxt = c2
            w = wb3[j]
            candj = jnp.min(jnp.where(w == mn, lane_c, big_i),
                            axis=1, keepdims=True)
            local_win = jnp.logical_and(cur == big_i, candj < big_i)
            clearmask = jnp.logical_and(local_win, lane_c == candj)
            neww = jnp.where(clearmask, jnp.float32(1e30), w)
            wb3[j] = neww
            cur = jnp.where(local_win, candj + j * NB, cur)
            nxt = jnp.minimum(nxt, jnp.min(neww, axis=1, keepdims=True))
            return cur, nxt

        sel, nmn = lax.fori_loop(
            0, NCH, sweep,
            (jnp.full((BB, 1), big_i),
             jnp.full((BB, 1), jnp.inf, jnp.float32)))
        return nmn, jnp.where(lane_k == k, sel, sels)

    _, sels = lax.fori_loop(0, K, kbody,
                            (mn0, jnp.zeros((BB, 128), jnp.int32)))
    idx_ref[...] = sels


_NC, _NS = 2, 16
_NW = _NC * _NS
_QPW = B // _NW
_IPW = _QPW * K
_ICH = _IPW // 128


def _sc_gather_body(q_hbm, idx_hbm, out_hbm, idx_v, rows_v, acc_v, sem):
    wid = lax.axis_index("s") * _NC + lax.axis_index("c")
    for j in range(_ICH):
        pltpu.sync_copy(idx_hbm.at[pl.ds(wid * _IPW + j * 128, 128)],
                        idx_v.at[j])
    copies = [
        pltpu.async_copy(q_hbm.at[idx_v.at[j]],
                         rows_v.at[pl.ds(j * 128, 128)], sem)
        for j in range(_ICH)
    ]
    for cp in copies:
        cp.wait()

    def qbody(i, _):
        a0 = jnp.zeros((16,), jnp.float32)
        a1 = jnp.zeros((16,), jnp.float32)
        for j in range(K):
            a0 = a0 + rows_v[i * K + j, pl.ds(0, 16)]
            a1 = a1 + rows_v[i * K + j, pl.ds(16, 16)]
        acc_v[i, pl.ds(0, 16)] = a0
        acc_v[i, pl.ds(16, 16)] = a1
        z16 = jnp.zeros((16,), jnp.float32)
        for h in range(2, 8):
            acc_v[i, pl.ds(h * 16, 16)] = z16
        return 0

    lax.fori_loop(0, _QPW, qbody, 0)
    pltpu.sync_copy(acc_v, out_hbm.at[pl.ds(wid * _QPW, _QPW)])


@functools.cache
def _build_sc_gather():
    return functools.partial(
        pl.kernel,
        out_type=jax.ShapeDtypeStruct((B, DP), jnp.float32),
        mesh=plsc.VectorSubcoreMesh(core_axis_name="c", subcore_axis_name="s"),
        scratch_types=[
            pltpu.VMEM((_ICH, 128), jnp.int32),
            pltpu.VMEM((_IPW, DP), jnp.float32),
            pltpu.VMEM((_QPW, DP), jnp.float32),
            pltpu.SemaphoreType.DMA,
        ],
    )(_sc_gather_body)


def kernel(z, supports, labels, weight, alpha_be, gamma_be, ensemble_bias):
    f32 = jnp.float32
    zp = jnp.pad(z.astype(f32), ((0, 0), (0, DP - D)))
    supp = jnp.pad(supports.astype(f32), ((0, NP - N), (0, DP - D)))
    labp = jnp.pad(labels.astype(f32), ((0, NP - N), (0, CP - C)))
    wp = jnp.pad(weight.astype(f32), ((0, DP - D), (0, DP - D)))
    ap = jnp.pad(alpha_be.astype(f32), ((0, 8 - E), (0, DP - D)))
    gp = jnp.pad(gamma_be.astype(f32), ((0, 8 - E), (0, DP - D)))
    bp = jnp.pad(ensemble_bias.astype(f32), ((0, 8 - E), (0, DP - D)))

    grid1 = (NP // NA,)
    full = lambda shape: pl.BlockSpec(shape, lambda i: (0, 0))
    u, cnt = pl.pallas_call(
        _stage1a_body,
        grid=grid1,
        in_specs=[
            pl.BlockSpec((NA, DP), lambda i: (i, 0)),
            pl.BlockSpec((NA, CP), lambda i: (i, 0)),
            full((8, DP)), full((8, DP)), full((8, DP)), full((DP, DP)),
        ],
        out_specs=(full((E * CP, DP)), full((CP, DP))),
        out_shape=(jax.ShapeDtypeStruct((E * CP, DP), f32),
                   jax.ShapeDtypeStruct((CP, DP), f32)),
    )(supp, labp, ap, gp, bp, wp)

    q, ya = pl.pallas_call(
        _stage1b_body,
        grid=grid1,
        in_specs=[
            pl.BlockSpec((NA, DP), lambda i: (i, 0)),
            full((8, DP)), full((8, DP)), full((8, DP)), full((DP, DP)),
            full((E * CP, DP)), full((CP, DP)),
        ],
        out_specs=(pl.BlockSpec((NA, DP), lambda i: (i, 0)),
                   pl.BlockSpec((NA, DP), lambda i: (i, 0))),
        out_shape=(jax.ShapeDtypeStruct((NP, DP), f32),
                   jax.ShapeDtypeStruct((NP, DP), f32)),
    )(supp, ap, gp, bp, wp, u, cnt)

    idx = pl.pallas_call(
        _stage2_body,
        grid=(B // BB,),
        in_specs=[
            pl.BlockSpec((BB, DP), lambda i: (i, 0)),
            pl.BlockSpec((NP, DP), lambda i: (0, 0)),
        ],
        out_specs=pl.BlockSpec((BB, 128), lambda i: (i, 0)),
        out_shape=jax.ShapeDtypeStruct((B, 128), jnp.int32),
        scratch_shapes=[pltpu.VMEM((NCH, BB, NB), f32)],
    )(zp, ya)

    idx_flat = idx[:, :K].reshape(B * K)
    out = _build_sc_gather()(q, idx_flat)
    return out[:, :C]


# speedup vs baseline: 1.9704x; 1.0700x over previous
import functools

import jax
import jax.numpy as jnp
from jax import lax
from jax.experimental import pallas as pl
from jax.experimental.pallas import tpu as pltpu
from jax.experimental.pallas import tpu_sc as plsc

B, N, D, C, E = 1024, 20000, 64, 17, 5
TAU, K = 10.0, 20
DP = 128
CP = 32
NP = 20480
NA = 2048
BB = 256
NEG_BIG = -1e30


def _rownorm(x):
    n = jnp.sqrt(jnp.sum(x * x, axis=1, keepdims=True))
    return x / jnp.maximum(n, 1e-12)


def _stage1a_body(sup_ref, lab_ref, alpha_ref, gamma_ref, bias_ref, w_ref,
                  u_ref, cnt_ref):
    @pl.when(pl.program_id(0) == 0)
    def _init():
        u_ref[...] = jnp.zeros_like(u_ref)
        cnt_ref[...] = jnp.zeros_like(cnt_ref)

    sup = sup_ref[...]
    lab = lab_ref[...]
    w = w_ref[...]
    ones = jnp.ones((NA, DP), jnp.float32)
    cnt_ref[...] += lax.dot_general(lab, ones, (((0,), (0,)), ((), ())),
                                    preferred_element_type=jnp.float32)
    for e in range(E):
        r = sup * alpha_ref[e:e + 1, :]
        mlp = lax.dot_general(r, w, (((1,), (1,)), ((), ())),
                              preferred_element_type=jnp.float32)
        mlp = mlp * gamma_ref[e:e + 1, :] + bias_ref[e:e + 1, :]
        u_ref[e * CP:(e + 1) * CP, :] += lax.dot_general(
            lab, mlp, (((0,), (0,)), ((), ())),
            preferred_element_type=jnp.float32)


def _stage1b_body(sup_ref, alpha_ref, gamma_ref, bias_ref, w_ref,
                  u_ref, cnt_ref, q_ref, ya_ref):
    sup = sup_ref[...]
    w = w_ref[...]
    yn = _rownorm(sup)
    yy = jnp.sum(yn * yn, axis=1, keepdims=True)
    rowid = pl.program_id(0) * NA + lax.broadcasted_iota(jnp.int32, (NA, 1), 0)
    yy = jnp.where(rowid < N, yy, 1e30)
    lane_y = lax.broadcasted_iota(jnp.int32, (NA, DP), 1)
    ya_ref[...] = jnp.where(lane_y == D, yy, yn)
    lane_c = lax.broadcasted_iota(jnp.int32, (NA, CP), 1)
    qacc = jnp.zeros((NA, CP), jnp.float32)
    for e in range(E):
        r = sup * alpha_ref[e:e + 1, :]
        mlp = lax.dot_general(r, w, (((1,), (1,)), ((), ())),
                              preferred_element_type=jnp.float32)
        mlp = mlp * gamma_ref[e:e + 1, :] + bias_ref[e:e + 1, :]
        tz = _rownorm(mlp)
        cen = u_ref[e * CP:(e + 1) * CP, :] / (cnt_ref[...] + 1e-12)
        cenn = _rownorm(cen)
        logits = TAU * lax.dot_general(tz, cenn, (((1,), (1,)), ((), ())),
                                       preferred_element_type=jnp.float32)
        logits = jnp.where(lane_c < C, logits, NEG_BIG)
        m = jnp.max(logits, axis=1, keepdims=True)
        p = jnp.exp(logits - m)
        qacc = qacc + p / jnp.sum(p, axis=1, keepdims=True)
    q_ref[...] = jnp.concatenate(
        [qacc * (1.0 / (E * K)), jnp.zeros((NA, DP - CP), jnp.float32)],
        axis=1)


NCH = 2
NB = NP // NCH


def _stage2_body(z_ref, ya_ref, idx_ref, wb3):
    zb = z_ref[...]
    x = _rownorm(zb)
    lane_d = lax.broadcasted_iota(jnp.int32, (BB, DP), 1)
    xa = jnp.where(lane_d == D, 1.0, -2.0 * x)
    for j in range(NCH):
        for t in range(NB // NA):
            wb3[j, :, t * NA:(t + 1) * NA] = lax.dot_general(
                xa, ya_ref[j * NB + t * NA:j * NB + (t + 1) * NA, :],
                (((1,), (1,)), ((), ())),
                preferred_element_type=jnp.float32)

    lane_c = lax.broadcasted_iota(jnp.int32, (BB, NB), 1)
    lane_k = lax.broadcasted_iota(jnp.int32, (BB, 128), 1)
    big_i = jnp.int32(2**30)

    def mn_body(j, cur):
        return jnp.minimum(cur, jnp.min(wb3[j], axis=1, keepdims=True))
    mn0 = lax.fori_loop(0, NCH, mn_body,
                        jnp.full((BB, 1), jnp.inf, jnp.float32))

    def kbody(k, carry):
        mn, sels = carry

        def sweep(j, c2):
            cur, nxt = c2
            w = wb3[j]
            candj = jnp.min(jnp.where(w == mn, lane_c, big_i),
                            axis=1, keepdims=True)
            local_win = jnp.logical_and(cur == big_i, candj < big_i)
            clearmask = jnp.logical_and(local_win, lane_c == candj)
            neww = jnp.where(clearmask, jnp.float32(1e30), w)
            wb3[j] = neww
            cur = jnp.where(local_win, candj + j * NB, cur)
            nxt = jnp.minimum(nxt, jnp.min(neww, axis=1, keepdims=True))
            return cur, nxt

        sel, nmn = lax.fori_loop(
            0, NCH, sweep,
            (jnp.full((BB, 1), big_i),
             jnp.full((BB, 1), jnp.inf, jnp.float32)))
        return nmn, jnp.where(lane_k == k, sel, sels)

    _, sels = lax.fori_loop(0, K, kbody,
                            (mn0, jnp.zeros((BB, 128), jnp.int32)))
    idx_ref[...] = sels


_NC, _NS = 2, 16
_NW = _NC * _NS
_QPW = B // _NW
_IPW = _QPW * K
_ICH = _IPW // 128


def _sc_gather_body(q_hbm, idx_hbm, out_hbm, idx_v, rows_v, acc_v, sem):
    wid = lax.axis_index("s") * _NC + lax.axis_index("c")
    for j in range(_ICH):
        pltpu.sync_copy(idx_hbm.at[pl.ds(wid * _IPW + j * 128, 128)],
                        idx_v.at[j])
    copies = [
        pltpu.async_copy(q_hbm.at[idx_v.at[j]],
                         rows_v.at[pl.ds(j * 128, 128)], sem)
        for j in range(_ICH)
    ]
    for cp in copies:
        cp.wait()

    def qbody(i, _):
        a0 = jnp.zeros((16,), jnp.float32)
        a1 = jnp.zeros((16,), jnp.float32)
        for j in range(K):
            a0 = a0 + rows_v[i * K + j, pl.ds(0, 16)]
            a1 = a1 + rows_v[i * K + j, pl.ds(16, 16)]
        acc_v[i, pl.ds(0, 16)] = a0
        acc_v[i, pl.ds(16, 16)] = a1
        z16 = jnp.zeros((16,), jnp.float32)
        for h in range(2, 8):
            acc_v[i, pl.ds(h * 16, 16)] = z16
        return 0

    lax.fori_loop(0, _QPW, qbody, 0)
    pltpu.sync_copy(acc_v, out_hbm.at[pl.ds(wid * _QPW, _QPW)])


@functools.cache
def _build_sc_gather():
    return functools.partial(
        pl.kernel,
        out_type=jax.ShapeDtypeStruct((B, DP), jnp.float32),
        mesh=plsc.VectorSubcoreMesh(core_axis_name="c", subcore_axis_name="s"),
        scratch_types=[
            pltpu.VMEM((_ICH, 128), jnp.int32),
            pltpu.VMEM((_IPW, DP), jnp.float32),
            pltpu.VMEM((_QPW, DP), jnp.float32),
            pltpu.SemaphoreType.DMA,
        ],
    )(_sc_gather_body)


def kernel(z, supports, labels, weight, alpha_be, gamma_be, ensemble_bias):
    f32 = jnp.float32
    zp = jnp.pad(z.astype(f32), ((0, 0), (0, DP - D)))
    supp = jnp.pad(supports.astype(f32), ((0, NP - N), (0, DP - D)))
    labp = jnp.pad(labels.astype(f32), ((0, NP - N), (0, CP - C)))
    wp = jnp.pad(weight.astype(f32), ((0, DP - D), (0, DP - D)))
    ap = jnp.pad(alpha_be.astype(f32), ((0, 8 - E), (0, DP - D)))
    gp = jnp.pad(gamma_be.astype(f32), ((0, 8 - E), (0, DP - D)))
    bp = jnp.pad(ensemble_bias.astype(f32), ((0, 8 - E), (0, DP - D)))

    grid1 = (NP // NA,)
    full = lambda shape: pl.BlockSpec(shape, lambda i: (0, 0))
    u, cnt = pl.pallas_call(
        _stage1a_body,
        grid=grid1,
        in_specs=[
            pl.BlockSpec((NA, DP), lambda i: (i, 0)),
            pl.BlockSpec((NA, CP), lambda i: (i, 0)),
            full((8, DP)), full((8, DP)), full((8, DP)), full((DP, DP)),
        ],
        out_specs=(full((E * CP, DP)), full((CP, DP))),
        out_shape=(jax.ShapeDtypeStruct((E * CP, DP), f32),
                   jax.ShapeDtypeStruct((CP, DP), f32)),
    )(supp, labp, ap, gp, bp, wp)

    q, ya = pl.pallas_call(
        _stage1b_body,
        grid=grid1,
        in_specs=[
            pl.BlockSpec((NA, DP), lambda i: (i, 0)),
            full((8, DP)), full((8, DP)), full((8, DP)), full((DP, DP)),
            full((E * CP, DP)), full((CP, DP)),
        ],
        out_specs=(pl.BlockSpec((NA, DP), lambda i: (i, 0)),
                   pl.BlockSpec((NA, DP), lambda i: (i, 0))),
        out_shape=(jax.ShapeDtypeStruct((NP, DP), f32),
                   jax.ShapeDtypeStruct((NP, DP), f32)),
    )(supp, ap, gp, bp, wp, u, cnt)

    idx = pl.pallas_call(
        _stage2_body,
        grid=(B // BB,),
        in_specs=[
            pl.BlockSpec((BB, DP), lambda i: (i, 0)),
            pl.BlockSpec((NP, DP), lambda i: (0, 0)),
        ],
        out_specs=pl.BlockSpec((BB, 128), lambda i: (i, 0)),
        out_shape=jax.ShapeDtypeStruct((B, 128), jnp.int32),
        scratch_shapes=[pltpu.VMEM((NCH, BB, NB), f32)],
    )(zp, ya)

    idx_flat = idx[:, :K].reshape(B * K)
    out = _build_sc_gather()(q, idx_flat)
    return out[:, :C]
